# baseline (speedup 1.0000x reference)
import functools

import jax
import jax.numpy as jnp
from jax import lax
from jax.experimental import pallas as pl
from jax.experimental.pallas import tpu as pltpu
from jax.experimental.pallas import tpu_sc as plsc

_EPS = 0.1
_NC = 100000
_INV_EPS = _EPS / (_NC - 1)
_B = 1024
_COEF = (1.0 - _EPS) - _INV_EPS

_V0 = 52800

_BLK_V = 6600


def _sum_body(x_ref, o_ref):
    @pl.when(pl.program_id(0) == 0)
    def _init():
        o_ref[0, 0] = 0.0

    o_ref[0, 0] += jnp.sum(x_ref[...])


def _dense_sum(xT):
    return pl.pallas_call(
        _sum_body,
        grid=(_V0 // _BLK_V,),
        in_specs=[pl.BlockSpec((_BLK_V, _B), lambda i: (i, 0))],
        out_specs=pl.BlockSpec((1, 1), lambda i: (0, 0), memory_space=pltpu.SMEM),
        out_shape=jax.ShapeDtypeStruct((1, 1), jnp.float32),
    )(xT)


_info = plsc.get_sparse_core_info()
_NCORES = _info.num_cores
_NSUB = _info.num_subcores
_NW = _NCORES * _NSUB
_RPW = _B // _NW
_L = 16

_TW = 128
_TH = 8

_CH = 32
_NCHUNK = (_NC - _V0) // _CH
_TSUB = -(-_NCHUNK // _NW)


def _sc_body(predT_hbm, tgt_hbm, out_hbm, tgt_v, win_v, part_v, buf0, buf1,
             gsem, sem0, sem1):
    wid = lax.axis_index("s") * _NCORES + lax.axis_index("c")
    base = wid * _RPW
    pltpu.sync_copy(tgt_hbm.at[pl.ds(base, _RPW)], tgt_v)
    iota = lax.iota(jnp.int32, _L)
    copies = []
    scalars = []
    tvecs = [tgt_v[pl.ds(c * _L, _L)] for c in range(_RPW // _L)]
    for r in range(_RPW):
        t = tvecs[r // _L][r % _L]
        v0 = pl.multiple_of(jnp.bitwise_and(t, -_TH), _TH)
        b0 = pl.multiple_of(jnp.bitwise_and(jnp.int32(base + r), -_TW), _TW)
        copies.append(
            pltpu.async_copy(
                predT_hbm.at[pl.ds(v0, _TH), pl.ds(b0, _TW)],
                win_v.at[pl.ds(r * _TH, _TH), :],
                gsem,
            )
        )
        scalars.append(t)

    def _src(j):
        row = _V0 + _CH * (wid + _NW * j)
        row = pl.multiple_of(jnp.minimum(row, _NC - _CH), _TH)
        return predT_hbm.at[pl.ds(row, _CH), :]

    def _acc_chunk(accs, buf, valid):
        def jbody(j, a):
            a0, a1, a2, a3 = a
            col = j * 32
            for rr in range(_CH // 4):
                for h in range(2):
                    d = pl.ds(col + h * _L, _L)
                    a0 = a0 + buf[4 * rr + 0, d]
                    a1 = a1 + buf[4 * rr + 1, d]
                    a2 = a2 + buf[4 * rr + 2, d]
                    a3 = a3 + buf[4 * rr + 3, d]
            return (a0, a1, a2, a3)

        z = jnp.zeros((_L,), jnp.float32)
        d0, d1, d2, d3 = lax.fori_loop(0, _B // 32, jbody, (z, z, z, z))
        a0, a1, a2, a3 = accs
        return (a0 + valid * d0, a1 + valid * d1, a2 + valid * d2, a3 + valid * d3)

    def _valid(j):
        return jnp.where(wid + _NW * j < _NCHUNK, 1.0, 0.0).astype(jnp.float32)

    pltpu.async_copy(_src(jnp.int32(0)), buf0, sem0)

    def obody(i, accs):
        j0 = 2 * i
        j1 = j0 + 1
        pltpu.async_copy(_src(j1), buf1, sem1)
        pltpu.make_async_copy(_src(jnp.int32(0)), buf0, sem0).wait()
        accs = _acc_chunk(accs, buf0, _valid(j0))
        pltpu.async_copy(_src(j0 + 2), buf0, sem0)
        pltpu.make_async_copy(_src(jnp.int32(0)), buf1, sem1).wait()
        return _acc_chunk(accs, buf1, _valid(j1))

    z = jnp.zeros((_L,), jnp.float32)
    a0, a1, a2, a3 = lax.fori_loop(0, (_TSUB + 1) // 2, obody, (z, z, z, z))
    pltpu.make_async_copy(_src(jnp.int32(0)), buf0, sem0).wait()
    dense = (a0 + a1) + (a2 + a3)

    for cp in copies:
        cp.wait()
    gacc = jnp.zeros((_L,), jnp.float32)
    for r in range(_RPW):
        t = scalars[r]
        rowin = jnp.bitwise_and(t, _TH - 1)
        lane = jnp.bitwise_and(base + r, _TW - 1)
        chunk = jnp.bitwise_and(lane, -_L)
        v16 = win_v[r * _TH + rowin, pl.ds(chunk, _L)]
        gacc = gacc + jnp.where(iota == lane - chunk, v16, 0.0)
    part_v[...] = jnp.float32(_INV_EPS) * dense + jnp.float32(_COEF) * gacc
    pltpu.sync_copy(part_v, out_hbm.at[wid])


_sc_part = functools.partial(
    pl.kernel,
    mesh=plsc.VectorSubcoreMesh(core_axis_name="c", subcore_axis_name="s"),
    out_type=jax.ShapeDtypeStruct((_NW, _L), jnp.float32),
    scratch_types=[
        pltpu.VMEM((_RPW,), jnp.int32),
        pltpu.VMEM((_RPW * _TH, _TW), jnp.float32),
        pltpu.VMEM((_L,), jnp.float32),
        pltpu.VMEM((_CH, _B), jnp.float32),
        pltpu.VMEM((_CH, _B), jnp.float32),
        pltpu.SemaphoreType.DMA,
        pltpu.SemaphoreType.DMA,
        pltpu.SemaphoreType.DMA,
    ],
)(_sc_body)


def kernel(predictions, targets):
    predT = predictions.T
    total = _dense_sum(predT)[0, 0]
    parts = _sc_part(predT, targets)
    return -(_INV_EPS * total + jnp.sum(parts)) / _B


# speedup vs baseline: 1.0467x; 1.0467x over previous
import functools

import jax
import jax.numpy as jnp
from jax import lax
from jax.experimental import pallas as pl
from jax.experimental.pallas import tpu as pltpu
from jax.experimental.pallas import tpu_sc as plsc

_EPS = 0.1
_NC = 100000
_INV_EPS = _EPS / (_NC - 1)
_B = 1024
_COEF = (1.0 - _EPS) - _INV_EPS

_V0 = 51200

_BLK_V = 6400


def _sum_body(x_ref, o_ref):
    @pl.when(pl.program_id(0) == 0)
    def _init():
        o_ref[0, 0] = 0.0

    o_ref[0, 0] += jnp.sum(x_ref[...])


def _dense_sum(xT):
    return pl.pallas_call(
        _sum_body,
        grid=(_V0 // _BLK_V,),
        in_specs=[pl.BlockSpec((_BLK_V, _B), lambda i: (i, 0))],
        out_specs=pl.BlockSpec((1, 1), lambda i: (0, 0), memory_space=pltpu.SMEM),
        out_shape=jax.ShapeDtypeStruct((1, 1), jnp.float32),
    )(xT)


_info = plsc.get_sparse_core_info()
_NCORES = _info.num_cores
_NSUB = _info.num_subcores
_NW = _NCORES * _NSUB
_RPW = _B // _NW
_L = 16

_TW = 128
_TH = 8

_CH = 32
_NCHUNK = (_NC - _V0) // _CH
_TSUB = -(-_NCHUNK // _NW)


def _sc_body(predT_hbm, tgt_hbm, out_hbm, tgt_v, win_v, part_v, buf0, buf1,
             gsem, sem0, sem1):
    wid = lax.axis_index("s") * _NCORES + lax.axis_index("c")
    base = wid * _RPW
    pltpu.sync_copy(tgt_hbm.at[pl.ds(base, _RPW)], tgt_v)
    iota = lax.iota(jnp.int32, _L)
    copies = []
    scalars = []
    tvecs = [tgt_v[pl.ds(c * _L, _L)] for c in range(_RPW // _L)]
    for r in range(_RPW):
        t = tvecs[r // _L][r % _L]
        v0 = pl.multiple_of(jnp.bitwise_and(t, -_TH), _TH)
        b0 = pl.multiple_of(jnp.bitwise_and(jnp.int32(base + r), -_TW), _TW)
        copies.append(
            pltpu.async_copy(
                predT_hbm.at[pl.ds(v0, _TH), pl.ds(b0, _TW)],
                win_v.at[pl.ds(r * _TH, _TH), :],
                gsem,
            )
        )
        scalars.append(t)

    def _src(j):
        row = _V0 + _CH * (wid + _NW * j)
        row = pl.multiple_of(jnp.minimum(row, _NC - _CH), _TH)
        return predT_hbm.at[pl.ds(row, _CH), :]

    def _acc_chunk(accs, buf, valid):
        def jbody(j, a):
            a0, a1, a2, a3 = a
            col = j * 32
            for rr in range(_CH // 4):
                for h in range(2):
                    d = pl.ds(col + h * _L, _L)
                    a0 = a0 + buf[4 * rr + 0, d]
                    a1 = a1 + buf[4 * rr + 1, d]
                    a2 = a2 + buf[4 * rr + 2, d]
                    a3 = a3 + buf[4 * rr + 3, d]
            return (a0, a1, a2, a3)

        z = jnp.zeros((_L,), jnp.float32)
        d0, d1, d2, d3 = lax.fori_loop(0, _B // 32, jbody, (z, z, z, z))
        a0, a1, a2, a3 = accs
        return (a0 + valid * d0, a1 + valid * d1, a2 + valid * d2, a3 + valid * d3)

    def _valid(j):
        return jnp.where(wid + _NW * j < _NCHUNK, 1.0, 0.0).astype(jnp.float32)

    pltpu.async_copy(_src(jnp.int32(0)), buf0, sem0)

    def obody(i, accs):
        j0 = 2 * i
        j1 = j0 + 1
        pltpu.async_copy(_src(j1), buf1, sem1)
        pltpu.make_async_copy(_src(jnp.int32(0)), buf0, sem0).wait()
        accs = _acc_chunk(accs, buf0, _valid(j0))
        pltpu.async_copy(_src(j0 + 2), buf0, sem0)
        pltpu.make_async_copy(_src(jnp.int32(0)), buf1, sem1).wait()
        return _acc_chunk(accs, buf1, _valid(j1))

    z = jnp.zeros((_L,), jnp.float32)
    a0, a1, a2, a3 = lax.fori_loop(0, (_TSUB + 1) // 2, obody, (z, z, z, z))
    pltpu.make_async_copy(_src(jnp.int32(0)), buf0, sem0).wait()
    dense = (a0 + a1) + (a2 + a3)

    for cp in copies:
        cp.wait()
    gacc = jnp.zeros((_L,), jnp.float32)
    for r in range(_RPW):
        t = scalars[r]
        rowin = jnp.bitwise_and(t, _TH - 1)
        lane = jnp.bitwise_and(base + r, _TW - 1)
        chunk = jnp.bitwise_and(lane, -_L)
        v16 = win_v[r * _TH + rowin, pl.ds(chunk, _L)]
        gacc = gacc + jnp.where(iota == lane - chunk, v16, 0.0)
    part_v[...] = jnp.float32(_INV_EPS) * dense + jnp.float32(_COEF) * gacc
    pltpu.sync_copy(part_v, out_hbm.at[wid])


_sc_part = functools.partial(
    pl.kernel,
    mesh=plsc.VectorSubcoreMesh(core_axis_name="c", subcore_axis_name="s"),
    out_type=jax.ShapeDtypeStruct((_NW, _L), jnp.float32),
    scratch_types=[
        pltpu.VMEM((_RPW,), jnp.int32),
        pltpu.VMEM((_RPW * _TH, _TW), jnp.float32),
        pltpu.VMEM((_L,), jnp.float32),
        pltpu.VMEM((_CH, _B), jnp.float32),
        pltpu.VMEM((_CH, _B), jnp.float32),
        pltpu.SemaphoreType.DMA,
        pltpu.SemaphoreType.DMA,
        pltpu.SemaphoreType.DMA,
    ],
)(_sc_body)


def kernel(predictions, targets):
    predT = predictions.T
    total = _dense_sum(predT)[0, 0]
    parts = _sc_part(predT, targets)
    return -(_INV_EPS * total + jnp.sum(parts)) / _B
